# baseline (speedup 1.0000x reference)
import functools

import jax
import jax.numpy as jnp
from jax import lax
from jax.experimental import pallas as pl
from jax.experimental.pallas import tpu as pltpu
from jax.experimental.pallas import tpu_sc as plsc

NC = 2
NS = 16
NW = NC * NS
GRP = 100
GPB = 2
LANES = 16


def _make_pool(B, L, E):
  assert L == GPB * GRP
  assert B % NW == 0 and E % LANES == 0
  bpw = B // NW
  nch = E // LANES
  mesh = plsc.VectorSubcoreMesh(core_axis_name="c", subcore_axis_name="s")

  @functools.partial(
      pl.kernel,
      out_type=jax.ShapeDtypeStruct((B, E), jnp.float32),
      mesh=mesh,
      scratch_types=[
          pltpu.VMEM((GPB * bpw, GRP), jnp.int32),
          pltpu.VMEM((L, E), jnp.float32),
          pltpu.VMEM((L, E), jnp.float32),
          pltpu.VMEM((bpw, E), jnp.float32),
          pltpu.SemaphoreType.DMA,
          pltpu.SemaphoreType.DMA,
      ],
      compiler_params=pltpu.CompilerParams(use_tc_tiling_on_sc=False),
  )
  def pool(text_hbm, table_hbm, out_hbm, idx_v, buf0, buf1, out_v, sem0, sem1):
    cid = lax.axis_index("c")
    sid = lax.axis_index("s")
    wid = sid * NC + cid
    pltpu.sync_copy(text_hbm.at[pl.ds(wid * GPB * bpw, GPB * bpw)], idx_v)

    bufs = (buf0, buf1)
    sems = (sem0, sem1)

    def fire(i, b):
      for g in range(GPB):
        pltpu.async_copy(
            table_hbm.at[idx_v.at[GPB * i + g]],
            bufs[b].at[pl.ds(g * GRP, GRP)],
            sems[b],
        )

    def drain(b):
      for g in range(GPB):
        pltpu.make_async_copy(
            table_hbm.at[idx_v.at[0]],
            bufs[b].at[pl.ds(g * GRP, GRP)],
            sems[b],
        ).wait()

    def accumulate(b, i):
      buf = bufs[b]

      def body(r, accs):
        out = []
        for p in range(2):
          row = 2 * r + p
          for c in range(nch):
            out.append(accs[p * nch + c] + buf[row, pl.ds(c * LANES, LANES)])
        return tuple(out)

      zero = jnp.zeros((LANES,), jnp.float32)
      accs = lax.fori_loop(0, L // 2, body, (zero,) * (2 * nch))
      scale = jnp.float32(1.0 / L)
      for c in range(nch):
        out_v[i, pl.ds(c * LANES, LANES)] = (accs[c] + accs[nch + c]) * scale

    fire(0, 0)
    fire(1, 1)

    @pl.loop(0, bpw, step=2)
    def _(i2):
      for b in range(2):
        i = i2 + b
        drain(b)
        accumulate(b, i)

        @pl.when(i + 2 < bpw)
        def _():
          fire(i + 2, b)

    pltpu.sync_copy(out_v, out_hbm.at[pl.ds(wid * bpw, bpw)])

  return pool


def _mlp(x, W1, b1, W2, b2):
  B, E = x.shape
  H = W1.shape[1]
  O = W2.shape[1]
  BM = 512

  def body(x_ref, w1_ref, b1_ref, w2_ref, b2_ref, o_ref):
    h = jnp.dot(x_ref[...], w1_ref[...], preferred_element_type=jnp.float32)
    h = jnp.maximum(h + b1_ref[...], 0.0)
    o = jnp.dot(h, w2_ref[...], preferred_element_type=jnp.float32)
    o_ref[...] = o + b2_ref[...]

  return pl.pallas_call(
      body,
      grid=(B // BM,),
      in_specs=[
          pl.BlockSpec((BM, E), lambda i: (i, 0)),
          pl.BlockSpec((E, H), lambda i: (0, 0)),
          pl.BlockSpec((1, H), lambda i: (0, 0)),
          pl.BlockSpec((H, O), lambda i: (0, 0)),
          pl.BlockSpec((1, O), lambda i: (0, 0)),
      ],
      out_specs=pl.BlockSpec((BM, O), lambda i: (i, 0)),
      out_shape=jax.ShapeDtypeStruct((B, O), jnp.float32),
  )(x, W1, b1.reshape(1, H), W2, b2.reshape(1, O))


def kernel(text, emb_table, W1, b1, W2, b2):
  B, L = text.shape
  _, E = emb_table.shape
  text2 = text.astype(jnp.int32).reshape(B * GPB, GRP)
  pooled = _make_pool(B, L, E)(text2, emb_table)
  return _mlp(pooled, W1, b1, W2, b2)
